# Initial kernel scaffold; baseline (speedup 1.0000x reference)
import jax
import jax.numpy as jnp
from jax.experimental import pallas as pl


def kernel(x_user, x_item, edge_index_user_rates_item, edge_index_item_rev_rates_user, Wk_ui, bk_ui, Wq_ui, bq_ui, Wv_ui, bv_ui, Wk_iu, bk_iu, Wq_iu, bq_iu, Wv_iu, bv_iu, Wo_user, bo_user, Wo_item, bo_item):
    raise NotImplementedError("write your pallas kernel here")


# speedup vs baseline: 1.0659x; 1.0659x over previous
import functools

import jax
import jax.numpy as jnp
from jax import lax
from jax.experimental import pallas as pl
from jax.experimental.pallas import tpu as pltpu
from jax.experimental.pallas import tpu_sc as plsc

N_NODE = 10000
E_EDGE = 160000
D_IN = 256
NHEAD = 4
CDIM = 64
HALF = 128

NCORE = 2
NSUB = 16
CHUNK = 128
CPT = 80
E_PAD = NSUB * CPT * CHUNK
AGG_ROWS = 10240
ROWS_PT = AGG_ROWS // NSUB

_MESH = plsc.VectorSubcoreMesh(core_axis_name="c", subcore_axis_name="s")
_SC_PARAMS = pltpu.CompilerParams(needs_layout_passes=False)


def _proj_body(xu, xi,
               wk_ui, bk_ui, wq_ui, bq_ui, wv_ui, bv_ui,
               wk_iu, bk_iu, wq_iu, bq_iu, wv_iu, bv_iu,
               kt_ui, qt_ui, vt_ui, kt_iu, qt_iu, vt_iu):
    xu_ = xu[...]
    xi_ = xi[...]

    def proj(x, w, b, out, scale):
        y = jnp.dot(x, w[...], preferred_element_type=jnp.float32)
        y = (y + b[...]) * scale
        out[0, :, :] = y[:, :HALF]
        out[1, :, :] = y[:, HALF:]

    proj(xu_, wk_ui, bk_ui, kt_ui, 0.125)
    proj(xi_, wq_ui, bq_ui, qt_ui, 1.0)
    proj(xu_, wv_ui, bv_ui, vt_ui, 1.0)
    proj(xi_, wk_iu, bk_iu, kt_iu, 0.125)
    proj(xu_, wq_iu, bq_iu, qt_iu, 1.0)
    proj(xi_, wv_iu, bv_iu, vt_iu, 1.0)


def _project(x_user, x_item, ws):
    nb = 10
    rows = N_NODE // nb
    x_spec = pl.BlockSpec((rows, D_IN), lambda i: (i, 0))
    w_spec = pl.BlockSpec((D_IN, D_IN), lambda i: (0, 0))
    b_spec = pl.BlockSpec((1, D_IN), lambda i: (0, 0))
    o_spec = pl.BlockSpec((2, rows, HALF), lambda i: (0, i, 0))
    out_shape = jax.ShapeDtypeStruct((2, N_NODE, HALF), jnp.float32)
    return pl.pallas_call(
        _proj_body,
        grid=(nb,),
        in_specs=[x_spec, x_spec] + [w_spec, b_spec] * 6,
        out_specs=[o_spec] * 6,
        out_shape=[out_shape] * 6,
    )(x_user, x_item, *ws)


@functools.partial(
    pl.kernel,
    out_type=jax.ShapeDtypeStruct((2 * 4 * E_PAD,), jnp.float32),
    mesh=_MESH,
    scratch_types=[
        pltpu.VMEM((2, CHUNK), jnp.int32),
        pltpu.VMEM((2, CHUNK), jnp.int32),
        pltpu.VMEM((2, CHUNK, HALF), jnp.float32),
        pltpu.VMEM((2, CHUNK, HALF), jnp.float32),
        pltpu.VMEM((2, 2, CHUNK), jnp.float32),
        pltpu.SemaphoreType.DMA,
        pltpu.SemaphoreType.DMA,
        pltpu.SemaphoreType.DMA,
    ],
    compiler_params=_SC_PARAMS,
)
def _scores_kernel(kt_ui, qt_ui, row_ui, col_ui,
                   kt_iu, qt_iu, row_iu, col_iu, out,
                   idxk, idxq, kbuf, qbuf, sbuf, sem0, sem1, sems):
    c = lax.axis_index("c")
    s = lax.axis_index("s")
    base_e = s * (CPT * CHUNK)
    offv = jnp.full((16,), c * N_NODE, jnp.int32)
    e_iota = lax.iota(jnp.int32, 16)
    sems_list = [sem0, sem1]

    def run_rel(kt, qt, row, col, out_base):
        def issue(t, j):
            e0 = base_e + t * CHUNK
            pltpu.sync_copy(row.at[pl.ds(e0, CHUNK)], idxk.at[j])
            pltpu.sync_copy(col.at[pl.ds(e0, CHUNK)], idxq.at[j])
            for i in range(CHUNK // 16):
                idxk[j, pl.ds(i * 16, 16)] = idxk[j, pl.ds(i * 16, 16)] + offv
                idxq[j, pl.ds(i * 16, 16)] = idxq[j, pl.ds(i * 16, 16)] + offv
            pltpu.async_copy(kt.at[idxk.at[j]], kbuf.at[j], sems_list[j])
            pltpu.async_copy(qt.at[idxq.at[j]], qbuf.at[j], sems_list[j])

        def wait(j):
            pltpu.make_async_copy(kt_ui.at[idxk.at[j]], kbuf.at[j],
                                  sems_list[j]).wait()
            pltpu.make_async_copy(qt_ui.at[idxq.at[j]], qbuf.at[j],
                                  sems_list[j]).wait()

        def compute(t, j):
            e0 = base_e + t * CHUNK
            jv = jnp.full((16,), j, jnp.int32)

            def group_body(g, carry2):
                eidx = e_iota + g * 16
                acc0 = jnp.zeros((16,), jnp.float32)
                acc1 = jnp.zeros((16,), jnp.float32)
                for cc in range(CDIM):
                    c0 = jnp.full((16,), cc, jnp.int32)
                    c1 = jnp.full((16,), CDIM + cc, jnp.int32)
                    acc0 = acc0 + (plsc.load_gather(kbuf, [jv, eidx, c0]) *
                                   plsc.load_gather(qbuf, [jv, eidx, c0]))
                    acc1 = acc1 + (plsc.load_gather(kbuf, [jv, eidx, c1]) *
                                   plsc.load_gather(qbuf, [jv, eidx, c1]))
                st = pl.multiple_of(g * 16, 16)
                sbuf[j, 0, pl.ds(st, 16)] = acc0
                sbuf[j, 1, pl.ds(st, 16)] = acc1
                return carry2

            lax.fori_loop(0, CHUNK // 16, group_body, 0)
            for h in range(2):
                pltpu.async_copy(
                    sbuf.at[j, h],
                    out.at[pl.ds(out_base + (2 * c + h) * E_PAD + e0, CHUNK)],
                    sems)

        issue(0, 0)

        def body(g, carry):
            t0 = 2 * g
            issue(t0 + 1, 1)
            wait(0)
            compute(t0, 0)

            @pl.when(t0 + 2 < CPT)
            def _():
                issue(t0 + 2, 0)

            wait(1)
            compute(t0 + 1, 1)
            return carry

        lax.fori_loop(0, CPT // 2, body, 0)

        def drain(i, carry):
            pltpu.make_async_copy(sbuf.at[0, 0], out.at[pl.ds(0, CHUNK)],
                                  sems).wait()
            return carry

        lax.fori_loop(0, 2 * CPT, drain, 0)

    run_rel(kt_ui, qt_ui, row_ui, col_ui, 0)
    run_rel(kt_iu, qt_iu, row_iu, col_iu, 4 * E_PAD)


def _softmax_body(su, si, au, ai):
    mask = lax.broadcasted_iota(jnp.int32, (4, E_PAD), 1) < E_EDGE

    def f(s_ref, a_ref):
        sc = s_ref[...]
        l = jnp.where(sc >= 0, sc, 0.2 * sc)
        l = jnp.where(mask, l, -1e30)
        m = jnp.max(l, axis=1, keepdims=True)
        p = jnp.where(mask, jnp.exp(l - m), 0.0)
        z = jnp.sum(p, axis=1, keepdims=True)
        a_ref[...] = p / z

    f(su, au)
    f(si, ai)


def _softmax(s_all):
    shp = jax.ShapeDtypeStruct((4, E_PAD), jnp.float32)
    return pl.pallas_call(
        _softmax_body,
        out_shape=[shp, shp],
    )(s_all[:4 * E_PAD].reshape(4, E_PAD), s_all[4 * E_PAD:].reshape(4, E_PAD))


@functools.partial(
    pl.kernel,
    out_type=jax.ShapeDtypeStruct((2 * AGG_ROWS, HALF), jnp.float32),
    mesh=_MESH,
    scratch_types=[
        pltpu.VMEM((2, CHUNK), jnp.int32),
        pltpu.VMEM((2, CHUNK), jnp.int32),
        pltpu.VMEM((2, CHUNK, HALF), jnp.float32),
        pltpu.VMEM((2, 2, CHUNK), jnp.float32),
        pltpu.VMEM_SHARED((AGG_ROWS, HALF), jnp.float32),
        pltpu.SemaphoreType.DMA,
        pltpu.SemaphoreType.DMA,
    ],
    compiler_params=_SC_PARAMS,
)
def _agg_kernel(vt, row, col, alpha, zeros, out,
                rowb, colb, vbuf, abuf, agg, sem0, sem1):
    c = lax.axis_index("c")
    s = lax.axis_index("s")
    base_e = s * (CPT * CHUNK)
    r0 = s * ROWS_PT
    offv = jnp.full((16,), c * N_NODE, jnp.int32)
    sems_list = [sem0, sem1]

    pltpu.sync_copy(zeros.at[pl.ds(r0, ROWS_PT)], agg.at[pl.ds(r0, ROWS_PT)])
    plsc.subcore_barrier()

    def issue(t, j):
        e0 = base_e + t * CHUNK
        pltpu.sync_copy(row.at[pl.ds(e0, CHUNK)], rowb.at[j])
        pltpu.sync_copy(col.at[pl.ds(e0, CHUNK)], colb.at[j])
        for i in range(CHUNK // 16):
            rowb[j, pl.ds(i * 16, 16)] = rowb[j, pl.ds(i * 16, 16)] + offv
        pltpu.async_copy(vt.at[rowb.at[j]], vbuf.at[j], sems_list[j])
        for h in range(2):
            pltpu.async_copy(
                alpha.at[pl.ds((2 * c + h) * E_PAD + e0, CHUNK)],
                abuf.at[j, h], sems_list[j])

    def wait(j):
        pltpu.make_async_copy(vt.at[rowb.at[j]], vbuf.at[j],
                              sems_list[j]).wait()
        for h in range(2):
            pltpu.make_async_copy(alpha.at[pl.ds(0, CHUNK)], abuf.at[j, h],
                                  sems_list[j]).wait()

    def compute(j):
        jv = jnp.full((16,), j, jnp.int32)

        def edge_body(e, carry2):
            ev = jnp.full((16,), e, jnp.int32)
            z16 = jnp.zeros((16,), jnp.int32)
            va0 = plsc.load_gather(abuf, [jv, z16, ev])
            va1 = plsc.load_gather(abuf, [jv, z16 + 1, ev])
            for jj in range(4):
                vbuf[j, e, pl.ds(jj * 16, 16)] = (
                    vbuf[j, e, pl.ds(jj * 16, 16)] * va0)
                vbuf[j, e, pl.ds(64 + jj * 16, 16)] = (
                    vbuf[j, e, pl.ds(64 + jj * 16, 16)] * va1)
            return carry2

        lax.fori_loop(0, CHUNK, edge_body, 0)
        pltpu.sync_copy(vbuf.at[j], agg.at[colb.at[j]], add=True)

    issue(0, 0)

    def body(g, carry):
        t0 = 2 * g
        issue(t0 + 1, 1)
        wait(0)
        compute(0)

        @pl.when(t0 + 2 < CPT)
        def _():
            issue(t0 + 2, 0)

        wait(1)
        compute(1)
        return carry

    lax.fori_loop(0, CPT // 2, body, 0)
    plsc.subcore_barrier()
    pltpu.sync_copy(agg.at[pl.ds(r0, ROWS_PT)],
                    out.at[pl.ds(c * AGG_ROWS + r0, ROWS_PT)])


def _out_body(agg_u, agg_i, wo_u, bo_u, wo_i, bo_i, out_u, out_i):
    def f(agg, wo, bo, out):
        lo = agg[pl.ds(0, N_NODE), :]
        hi = agg[pl.ds(AGG_ROWS, N_NODE), :]
        y = jnp.dot(lo, wo[pl.ds(0, HALF), :], preferred_element_type=jnp.float32)
        y = y + jnp.dot(hi, wo[pl.ds(HALF, HALF), :], preferred_element_type=jnp.float32)
        out[...] = y + bo[...]

    f(agg_u, wo_u, bo_u, out_u)
    f(agg_i, wo_i, bo_i, out_i)


def _output(agg_user, agg_item, wo_user, bo_user, wo_item, bo_item):
    shp = jax.ShapeDtypeStruct((N_NODE, CDIM), jnp.float32)
    return pl.pallas_call(
        _out_body,
        out_shape=[shp, shp],
    )(agg_user, agg_item, wo_user, bo_user.reshape(1, CDIM),
      wo_item, bo_item.reshape(1, CDIM))


def kernel(x_user, x_item, edge_index_user_rates_item, edge_index_item_rev_rates_user,
           Wk_ui, bk_ui, Wq_ui, bq_ui, Wv_ui, bv_ui,
           Wk_iu, bk_iu, Wq_iu, bq_iu, Wv_iu, bv_iu,
           Wo_user, bo_user, Wo_item, bo_item):
    pad = E_PAD - E_EDGE
    row_ui = jnp.pad(edge_index_user_rates_item[0], (0, pad))
    col_ui = jnp.pad(edge_index_user_rates_item[1], (0, pad))
    row_iu = jnp.pad(edge_index_item_rev_rates_user[0], (0, pad))
    col_iu = jnp.pad(edge_index_item_rev_rates_user[1], (0, pad))

    ws = [Wk_ui, bk_ui.reshape(1, -1), Wq_ui, bq_ui.reshape(1, -1),
          Wv_ui, bv_ui.reshape(1, -1), Wk_iu, bk_iu.reshape(1, -1),
          Wq_iu, bq_iu.reshape(1, -1), Wv_iu, bv_iu.reshape(1, -1)]
    kt_ui, qt_ui, vt_ui, kt_iu, qt_iu, vt_iu = _project(x_user, x_item, ws)
    kt_ui = kt_ui.reshape(2 * N_NODE, HALF)
    qt_ui = qt_ui.reshape(2 * N_NODE, HALF)
    vt_ui = vt_ui.reshape(2 * N_NODE, HALF)
    kt_iu = kt_iu.reshape(2 * N_NODE, HALF)
    qt_iu = qt_iu.reshape(2 * N_NODE, HALF)
    vt_iu = vt_iu.reshape(2 * N_NODE, HALF)

    s_all = _scores_kernel(kt_ui, qt_ui, row_ui, col_ui,
                           kt_iu, qt_iu, row_iu, col_iu)
    a_ui, a_iu = _softmax(s_all)

    zeros = jnp.zeros((AGG_ROWS, HALF), jnp.float32)
    agg_ui = _agg_kernel(vt_ui, row_ui, col_ui, a_ui.reshape(-1), zeros)
    agg_iu = _agg_kernel(vt_iu, row_iu, col_iu, a_iu.reshape(-1), zeros)

    out_user, out_item = _output(agg_iu, agg_ui, Wo_user, bo_user,
                                 Wo_item, bo_item)
    return (out_user, out_item)


# speedup vs baseline: 2.1769x; 2.0423x over previous
import functools

import jax
import jax.numpy as jnp
from jax import lax
from jax.experimental import pallas as pl
from jax.experimental.pallas import tpu as pltpu
from jax.experimental.pallas import tpu_sc as plsc

N_NODE = 10000
E_EDGE = 160000
D_IN = 256
NHEAD = 4
CDIM = 64
HALF = 128

NCORE = 2
NSUB = 16
CHUNK = 128
CPT = 80
E_PAD = NSUB * CPT * CHUNK
AGG_ROWS = 10240
ROWS_PT = AGG_ROWS // NSUB

_MESH = plsc.VectorSubcoreMesh(core_axis_name="c", subcore_axis_name="s")
_SC_PARAMS = pltpu.CompilerParams(needs_layout_passes=False)


def _proj_body(xu, xi,
               wk_ui, bk_ui, wq_ui, bq_ui, wv_ui, bv_ui,
               wk_iu, bk_iu, wq_iu, bq_iu, wv_iu, bv_iu,
               kt_ui, qt_ui, vt_ui, kt_iu, qt_iu, vt_iu):
    xu_ = xu[...]
    xi_ = xi[...]

    def proj(x, w, b, out, scale):
        y = jnp.dot(x, w[...], preferred_element_type=jnp.float32)
        y = (y + b[...]) * scale
        out[0, :, :] = y[:, :HALF]
        out[1, :, :] = y[:, HALF:]

    proj(xu_, wk_ui, bk_ui, kt_ui, 0.125)
    proj(xi_, wq_ui, bq_ui, qt_ui, 1.0)
    proj(xu_, wv_ui, bv_ui, vt_ui, 1.0)
    proj(xi_, wk_iu, bk_iu, kt_iu, 0.125)
    proj(xu_, wq_iu, bq_iu, qt_iu, 1.0)
    proj(xi_, wv_iu, bv_iu, vt_iu, 1.0)


def _project(x_user, x_item, ws):
    nb = 10
    rows = N_NODE // nb
    x_spec = pl.BlockSpec((rows, D_IN), lambda i: (i, 0))
    w_spec = pl.BlockSpec((D_IN, D_IN), lambda i: (0, 0))
    b_spec = pl.BlockSpec((1, D_IN), lambda i: (0, 0))
    o_spec = pl.BlockSpec((2, rows, HALF), lambda i: (0, i, 0))
    out_shape = jax.ShapeDtypeStruct((2, N_NODE, HALF), jnp.float32)
    return pl.pallas_call(
        _proj_body,
        grid=(nb,),
        in_specs=[x_spec, x_spec] + [w_spec, b_spec] * 6,
        out_specs=[o_spec] * 6,
        out_shape=[out_shape] * 6,
    )(x_user, x_item, *ws)


@functools.partial(
    pl.kernel,
    out_type=jax.ShapeDtypeStruct((2 * 4 * E_PAD,), jnp.float32),
    mesh=_MESH,
    scratch_types=[
        pltpu.VMEM((2, CHUNK), jnp.int32),
        pltpu.VMEM((2, CHUNK), jnp.int32),
        pltpu.VMEM((2, CHUNK, HALF), jnp.float32),
        pltpu.VMEM((2, CHUNK, HALF), jnp.float32),
        pltpu.VMEM((2, 2, CHUNK), jnp.float32),
        pltpu.SemaphoreType.DMA,
        pltpu.SemaphoreType.DMA,
        pltpu.SemaphoreType.DMA,
    ],
    compiler_params=_SC_PARAMS,
)
def _scores_kernel(kt_ui, qt_ui, row_ui, col_ui,
                   kt_iu, qt_iu, row_iu, col_iu, out,
                   idxk, idxq, kbuf, qbuf, sbuf, sem0, sem1, sems):
    c = lax.axis_index("c")
    s = lax.axis_index("s")
    base_e = s * (CPT * CHUNK)
    offv = jnp.full((16,), c * N_NODE, jnp.int32)
    e_iota = lax.iota(jnp.int32, 16)
    sems_list = [sem0, sem1]

    def run_rel(kt, qt, row, col, out_base):
        def issue(t, j):
            e0 = base_e + t * CHUNK
            pltpu.sync_copy(row.at[pl.ds(e0, CHUNK)], idxk.at[j])
            pltpu.sync_copy(col.at[pl.ds(e0, CHUNK)], idxq.at[j])
            for i in range(CHUNK // 16):
                idxk[j, pl.ds(i * 16, 16)] = idxk[j, pl.ds(i * 16, 16)] + offv
                idxq[j, pl.ds(i * 16, 16)] = idxq[j, pl.ds(i * 16, 16)] + offv
            pltpu.async_copy(kt.at[idxk.at[j]], kbuf.at[j], sems_list[j])
            pltpu.async_copy(qt.at[idxq.at[j]], qbuf.at[j], sems_list[j])

        def wait(j):
            pltpu.make_async_copy(kt_ui.at[idxk.at[j]], kbuf.at[j],
                                  sems_list[j]).wait()
            pltpu.make_async_copy(qt_ui.at[idxq.at[j]], qbuf.at[j],
                                  sems_list[j]).wait()

        def compute(t, j):
            e0 = base_e + t * CHUNK
            lane15 = e_iota == 15
            jv = jnp.full((16,), j, jnp.int32)
            z16 = jnp.zeros((16,), jnp.int32)

            def edge_body(eh, carry2):
                for u in range(2):
                    e = eh * 2 + u
                    acc0 = kbuf[j, e, pl.ds(0, 16)] * qbuf[j, e, pl.ds(0, 16)]
                    acc1 = (kbuf[j, e, pl.ds(CDIM, 16)] *
                            qbuf[j, e, pl.ds(CDIM, 16)])
                    for i in range(1, 4):
                        acc0 = acc0 + (kbuf[j, e, pl.ds(i * 16, 16)] *
                                       qbuf[j, e, pl.ds(i * 16, 16)])
                        acc1 = acc1 + (kbuf[j, e, pl.ds(CDIM + i * 16, 16)] *
                                       qbuf[j, e, pl.ds(CDIM + i * 16, 16)])
                    ev = jnp.full((16,), e, jnp.int32)
                    plsc.store_scatter(sbuf, [jv, z16, ev], plsc.cumsum(acc0),
                                       mask=lane15)
                    plsc.store_scatter(sbuf, [jv, z16 + 1, ev],
                                       plsc.cumsum(acc1), mask=lane15)
                return carry2

            lax.fori_loop(0, CHUNK // 2, edge_body, 0)
            for h in range(2):
                pltpu.async_copy(
                    sbuf.at[j, h],
                    out.at[pl.ds(out_base + (2 * c + h) * E_PAD + e0, CHUNK)],
                    sems)

        issue(0, 0)

        def body(g, carry):
            t0 = 2 * g
            issue(t0 + 1, 1)
            wait(0)
            compute(t0, 0)

            @pl.when(t0 + 2 < CPT)
            def _():
                issue(t0 + 2, 0)

            wait(1)
            compute(t0 + 1, 1)
            return carry

        lax.fori_loop(0, CPT // 2, body, 0)

        def drain(i, carry):
            pltpu.make_async_copy(sbuf.at[0, 0], out.at[pl.ds(0, CHUNK)],
                                  sems).wait()
            return carry

        lax.fori_loop(0, 2 * CPT, drain, 0)

    run_rel(kt_ui, qt_ui, row_ui, col_ui, 0)
    run_rel(kt_iu, qt_iu, row_iu, col_iu, 4 * E_PAD)


def _softmax_body(su, si, au, ai):
    mask = lax.broadcasted_iota(jnp.int32, (4, E_PAD), 1) < E_EDGE

    def f(s_ref, a_ref):
        sc = s_ref[...]
        l = jnp.where(sc >= 0, sc, 0.2 * sc)
        l = jnp.where(mask, l, -1e30)
        m = jnp.max(l, axis=1, keepdims=True)
        p = jnp.where(mask, jnp.exp(l - m), 0.0)
        z = jnp.sum(p, axis=1, keepdims=True)
        a_ref[...] = p / z

    f(su, au)
    f(si, ai)


def _softmax(s_all):
    shp = jax.ShapeDtypeStruct((4, E_PAD), jnp.float32)
    return pl.pallas_call(
        _softmax_body,
        out_shape=[shp, shp],
    )(s_all[:4 * E_PAD].reshape(4, E_PAD), s_all[4 * E_PAD:].reshape(4, E_PAD))


@functools.partial(
    pl.kernel,
    out_type=jax.ShapeDtypeStruct((4 * AGG_ROWS, HALF), jnp.float32),
    mesh=_MESH,
    scratch_types=[
        pltpu.VMEM((2, CHUNK), jnp.int32),
        pltpu.VMEM((2, CHUNK), jnp.int32),
        pltpu.VMEM((2, CHUNK, HALF), jnp.float32),
        pltpu.VMEM((2, 2, CHUNK), jnp.float32),
        pltpu.VMEM((64, HALF), jnp.float32),
        pltpu.VMEM_SHARED((AGG_ROWS, HALF), jnp.float32),
        pltpu.SemaphoreType.DMA,
        pltpu.SemaphoreType.DMA,
    ],
    compiler_params=_SC_PARAMS,
)
def _agg_kernel(vt_ui, row_ui, col_ui, a_ui, vt_iu, row_iu, col_iu, a_iu, out,
                rowb, colb, vbuf, abuf, zbuf, agg, sem0, sem1):
    c = lax.axis_index("c")
    s = lax.axis_index("s")
    base_e = s * (CPT * CHUNK)
    r0 = s * ROWS_PT
    offv = jnp.full((16,), c * N_NODE, jnp.int32)
    sems_list = [sem0, sem1]
    vzero = jnp.zeros((16,), jnp.float32)

    def zrow(e, carry):
        for i in range(HALF // 16):
            zbuf[e, pl.ds(i * 16, 16)] = vzero
        return carry

    lax.fori_loop(0, 64, zrow, 0)

    def run_rel(vt, row, col, alpha, out_base):
        def zblk(b, carry):
            pltpu.sync_copy(zbuf, agg.at[pl.ds(r0 + b * 64, 64)])
            return carry

        lax.fori_loop(0, ROWS_PT // 64, zblk, 0)
        plsc.subcore_barrier()

        def issue(t, j):
            e0 = base_e + t * CHUNK
            pltpu.sync_copy(row.at[pl.ds(e0, CHUNK)], rowb.at[j])
            pltpu.sync_copy(col.at[pl.ds(e0, CHUNK)], colb.at[j])
            for i in range(CHUNK // 16):
                rowb[j, pl.ds(i * 16, 16)] = rowb[j, pl.ds(i * 16, 16)] + offv
            pltpu.async_copy(vt.at[rowb.at[j]], vbuf.at[j], sems_list[j])
            for h in range(2):
                pltpu.async_copy(
                    alpha.at[pl.ds((2 * c + h) * E_PAD + e0, CHUNK)],
                    abuf.at[j, h], sems_list[j])

        def wait(j):
            pltpu.make_async_copy(vt_ui.at[rowb.at[j]], vbuf.at[j],
                                  sems_list[j]).wait()
            for h in range(2):
                pltpu.make_async_copy(a_ui.at[pl.ds(0, CHUNK)], abuf.at[j, h],
                                      sems_list[j]).wait()

        def compute(j):
            def edge_body(e, carry2):
                g16 = pl.multiple_of((e // 16) * 16, 16)
                em = jnp.full((16,), e - g16, jnp.int32)
                a0v = abuf[j, 0, pl.ds(g16, 16)]
                a1v = abuf[j, 1, pl.ds(g16, 16)]
                va0 = jnp.take_along_axis(a0v, em, axis=0)
                va1 = jnp.take_along_axis(a1v, em, axis=0)
                for jj in range(4):
                    vbuf[j, e, pl.ds(jj * 16, 16)] = (
                        vbuf[j, e, pl.ds(jj * 16, 16)] * va0)
                    vbuf[j, e, pl.ds(64 + jj * 16, 16)] = (
                        vbuf[j, e, pl.ds(64 + jj * 16, 16)] * va1)
                return carry2

            lax.fori_loop(0, CHUNK, edge_body, 0)
            pltpu.sync_copy(vbuf.at[j], agg.at[colb.at[j]], add=True)

        issue(0, 0)

        def body(g, carry):
            t0 = 2 * g
            issue(t0 + 1, 1)
            wait(0)
            compute(0)

            @pl.when(t0 + 2 < CPT)
            def _():
                issue(t0 + 2, 0)

            wait(1)
            compute(1)
            return carry

        lax.fori_loop(0, CPT // 2, body, 0)
        plsc.subcore_barrier()
        pltpu.sync_copy(agg.at[pl.ds(r0, ROWS_PT)],
                        out.at[pl.ds(out_base + c * AGG_ROWS + r0, ROWS_PT)])
        plsc.subcore_barrier()

    run_rel(vt_ui, row_ui, col_ui, a_ui, 0)
    run_rel(vt_iu, row_iu, col_iu, a_iu, 2 * AGG_ROWS)


def _out_body(agg_u, agg_i, wo_u, bo_u, wo_i, bo_i, out_u, out_i):
    def f(agg, wo, bo, out):
        lo = agg[pl.ds(0, N_NODE), :]
        hi = agg[pl.ds(AGG_ROWS, N_NODE), :]
        y = jnp.dot(lo, wo[pl.ds(0, HALF), :], preferred_element_type=jnp.float32)
        y = y + jnp.dot(hi, wo[pl.ds(HALF, HALF), :], preferred_element_type=jnp.float32)
        out[...] = y + bo[...]

    f(agg_u, wo_u, bo_u, out_u)
    f(agg_i, wo_i, bo_i, out_i)


def _output(agg_user, agg_item, wo_user, bo_user, wo_item, bo_item):
    shp = jax.ShapeDtypeStruct((N_NODE, CDIM), jnp.float32)
    return pl.pallas_call(
        _out_body,
        out_shape=[shp, shp],
    )(agg_user, agg_item, wo_user, bo_user.reshape(1, CDIM),
      wo_item, bo_item.reshape(1, CDIM))


def kernel(x_user, x_item, edge_index_user_rates_item, edge_index_item_rev_rates_user,
           Wk_ui, bk_ui, Wq_ui, bq_ui, Wv_ui, bv_ui,
           Wk_iu, bk_iu, Wq_iu, bq_iu, Wv_iu, bv_iu,
           Wo_user, bo_user, Wo_item, bo_item):
    pad = E_PAD - E_EDGE
    row_ui = jnp.pad(edge_index_user_rates_item[0], (0, pad))
    col_ui = jnp.pad(edge_index_user_rates_item[1], (0, pad))
    row_iu = jnp.pad(edge_index_item_rev_rates_user[0], (0, pad))
    col_iu = jnp.pad(edge_index_item_rev_rates_user[1], (0, pad))

    ws = [Wk_ui, bk_ui.reshape(1, -1), Wq_ui, bq_ui.reshape(1, -1),
          Wv_ui, bv_ui.reshape(1, -1), Wk_iu, bk_iu.reshape(1, -1),
          Wq_iu, bq_iu.reshape(1, -1), Wv_iu, bv_iu.reshape(1, -1)]
    kt_ui, qt_ui, vt_ui, kt_iu, qt_iu, vt_iu = _project(x_user, x_item, ws)
    kt_ui = kt_ui.reshape(2 * N_NODE, HALF)
    qt_ui = qt_ui.reshape(2 * N_NODE, HALF)
    vt_ui = vt_ui.reshape(2 * N_NODE, HALF)
    kt_iu = kt_iu.reshape(2 * N_NODE, HALF)
    qt_iu = qt_iu.reshape(2 * N_NODE, HALF)
    vt_iu = vt_iu.reshape(2 * N_NODE, HALF)

    s_all = _scores_kernel(kt_ui, qt_ui, row_ui, col_ui,
                           kt_iu, qt_iu, row_iu, col_iu)
    a_ui, a_iu = _softmax(s_all)

    agg_all = _agg_kernel(vt_ui, row_ui, col_ui, a_ui.reshape(-1),
                          vt_iu, row_iu, col_iu, a_iu.reshape(-1))
    agg_ui = agg_all[:2 * AGG_ROWS]
    agg_iu = agg_all[2 * AGG_ROWS:]

    out_user, out_item = _output(agg_iu, agg_ui, Wo_user, bo_user,
                                 Wo_item, bo_item)
    return (out_user, out_item)
